# baseline (speedup 1.0000x reference)
import functools

import jax
import jax.numpy as jnp
from jax import lax
from jax.experimental import pallas as pl
from jax.experimental.pallas import tpu as pltpu
from jax.experimental.pallas import tpu_sc as plsc

_B = 16384
_C = 1000
_NC = 2
_NS = 16
_NW = _NC * _NS
_L = 16
_BPW = _B // _NW
_W = 128
_NBKT = 8
_CHUNKS = _BPW // _L
_DG = 32
_NDMA = _BPW // _DG
_SEG = _BPW + _NBKT * (_DG - 1) + 8
_TAIL = _C - _W

_mesh = plsc.VectorSubcoreMesh(core_axis_name="c", subcore_axis_name="s")


@functools.partial(
    pl.kernel,
    mesh=_mesh,
    compiler_params=pltpu.CompilerParams(needs_layout_passes=False),
    out_type=jax.ShapeDtypeStruct((_B,), jnp.float32),
    scratch_types=[
        pltpu.VMEM((_BPW,), jnp.int32),
        pltpu.VMEM((_NBKT * _BPW,), jnp.int32),
        pltpu.VMEM((_BPW,), jnp.int32),
        pltpu.VMEM((_L,), jnp.int32),
        pltpu.VMEM((_L,), jnp.int32),
        pltpu.VMEM((_SEG, _W), jnp.float32),
        pltpu.VMEM((_BPW,), jnp.float32),
        pltpu.SemaphoreType.DMA,
    ],
)
def _depth_sc(x_hbm, xtail_hbm, labels_hbm, out_hbm, lab_v, idx_v, rank_v,
              cnt_v, gtab_v, seg_v, out_v, sem):
    wid = lax.axis_index("s") * _NC + lax.axis_index("c")
    base = wid * _BPW

    pltpu.sync_copy(labels_hbm.at[pl.ds(base, _BPW)], lab_v)

    lane = lax.iota(jnp.int32, _L)
    zero16 = jnp.full((_L,), 0, jnp.int32)

    cnt_v[pl.ds(0, _L)] = zero16

    def clear(i, _):
        for b in range(_NBKT):
            idx_v[pl.ds(b * _BPW + i * _L, _L)] = zero16
        return _

    with jax.named_scope("ph_clear"):
        lax.fori_loop(0, _CHUNKS, clear, None)

    def build(j, _):
        lab = lab_v[pl.ds(j * _L, _L)]
        bkt = jnp.where(lab >= _TAIL, _NBKT - 1, lax.shift_right_logical(lab, 7))
        rows = (base + j * _L) + lane
        occ, last = plsc.scan_count(bkt)
        nvec = plsc.load_gather(cnt_v, [bkt])
        pos = nvec + occ - 1
        plsc.store_scatter(idx_v, [bkt * _BPW + pos], rows)
        rank_v[pl.ds(j * _L, _L)] = pos
        plsc.addupdate_scatter(cnt_v, [bkt], occ, mask=last)
        return _

    with jax.named_scope("ph_build"):
        lax.fori_loop(0, _CHUNKS, build, None)

    counts16 = cnt_v[pl.ds(0, _L)]
    counts = [
        jnp.max(jnp.where(lane == b, counts16, 0)) for b in range(_NBKT)
    ]

    gbase = []
    acc = jnp.int32(0)
    for b in range(_NBKT):
        gbase.append(acc)
        acc = acc + ((counts[b] + _DG - 1) // _DG) * _DG
    gtab = zero16
    for b in range(_NBKT):
        gtab = jnp.where(lane == b, gbase[b], gtab)
    gtab_v[pl.ds(0, _L)] = gtab

    def dma(b, k):
        src_idx = plsc.Indices(idx_v.at[pl.ds(b * _BPW + k * _DG, _DG)])
        if b < _NBKT - 1:
            src = x_hbm.at[src_idx, pl.ds(b * _W, _W)]
        else:
            src = xtail_hbm.at[src_idx]
        dst = seg_v.at[pl.ds(gbase[b] + k * _DG, _DG), :]
        return pltpu.make_async_copy(src, dst, sem)

    for b in range(_NBKT):
        def start_k(k, _, b=b):
            @pl.when(k * _DG < counts[b])
            def _go():
                dma(b, k).start()
            return _
        with jax.named_scope("ph_start"):
            lax.fori_loop(0, _NDMA, start_k, None)
    for b in range(_NBKT):
        def wait_k(k, _, b=b):
            @pl.when(k * _DG < counts[b])
            def _go():
                dma(b, k).wait()
            return _
        with jax.named_scope("ph_wait"):
            lax.fori_loop(0, _NDMA, wait_k, None)

    def extract(j, _):
        lab = lab_v[pl.ds(j * _L, _L)]
        bkt = jnp.where(lab >= _TAIL, _NBKT - 1, lax.shift_right_logical(lab, 7))
        col = jnp.where(lab >= _TAIL, lab - _TAIL, lab & (_W - 1))
        pos = plsc.load_gather(gtab_v, [bkt]) + rank_v[pl.ds(j * _L, _L)]
        v = plsc.load_gather(seg_v, [pos, col])
        out_v[pl.ds(j * _L, _L)] = jnp.exp(jnp.abs(v * jnp.float32(0.1))) - 1.0
        return _

    with jax.named_scope("ph_extract"):
        lax.fori_loop(0, _CHUNKS, extract, None)

    pltpu.sync_copy(out_v, out_hbm.at[pl.ds(base, _BPW)])


def kernel(x, labels):
    depth = _depth_sc(x, x[:, _TAIL:], labels.astype(jnp.int32))
    return depth[:, None]


# speedup vs baseline: 4.7362x; 4.7362x over previous
import functools

import jax
import jax.numpy as jnp
from jax import lax
from jax.experimental import pallas as pl
from jax.experimental.pallas import tpu as pltpu
from jax.experimental.pallas import tpu_sc as plsc

_B = 16384
_C = 1000
_NC = 2
_NS = 16
_NW = _NC * _NS
_L = 16
_BPW = _B // _NW
_G = 128
_NG = _BPW // _G
_CHUNKS = _BPW // _L

_mesh = plsc.VectorSubcoreMesh(core_axis_name="c", subcore_axis_name="s")


@functools.partial(
    pl.kernel,
    mesh=_mesh,
    compiler_params=pltpu.CompilerParams(needs_layout_passes=False),
    out_type=jax.ShapeDtypeStruct((_B,), jnp.float32),
    scratch_types=[
        pltpu.VMEM((_BPW,), jnp.int32),
        pltpu.VMEM((_BPW, _G), jnp.float32),
        pltpu.VMEM((_BPW,), jnp.float32),
        pltpu.SemaphoreType.DMA,
    ],
)
def _depth_sc(xt_hbm, labels_hbm, out_hbm, lab_v, seg_v, out_v, sem):
    wid = lax.axis_index("s") * _NC + lax.axis_index("c")
    base = wid * _BPW

    pltpu.sync_copy(labels_hbm.at[pl.ds(base, _BPW)], lab_v)

    copies = [
        pltpu.make_async_copy(
            xt_hbm.at[
                plsc.Indices(lab_v.at[pl.ds(c * _G, _G)]),
                pl.ds(base + c * _G, _G),
            ],
            seg_v.at[pl.ds(c * _G, _G), :],
            sem,
        )
        for c in range(_NG)
    ]
    for cp in copies:
        cp.start()
    for cp in copies:
        cp.wait()

    lane = lax.iota(jnp.int32, _L)

    def extract(j, _):
        pos = j * _L + lane
        v = plsc.load_gather(seg_v, [pos, pos & (_G - 1)])
        out_v[pl.ds(j * _L, _L)] = jnp.exp(jnp.abs(v * jnp.float32(0.1))) - 1.0
        return _

    lax.fori_loop(0, _CHUNKS, extract, None)

    pltpu.sync_copy(out_v, out_hbm.at[pl.ds(base, _BPW)])


def kernel(x, labels):
    depth = _depth_sc(x.T, labels.astype(jnp.int32))
    return depth[:, None]


# speedup vs baseline: 4.7515x; 1.0032x over previous
import functools

import jax
import jax.numpy as jnp
from jax import lax
from jax.experimental import pallas as pl
from jax.experimental.pallas import tpu as pltpu
from jax.experimental.pallas import tpu_sc as plsc

_B = 16384
_C = 1000
_NC = 2
_NS = 16
_NW = _NC * _NS
_L = 16
_BPW = _B // _NW
_G = 128
_NG = _BPW // _G
_CHUNKS = _BPW // _L

_mesh = plsc.VectorSubcoreMesh(core_axis_name="c", subcore_axis_name="s")


@functools.partial(
    pl.kernel,
    mesh=_mesh,
    compiler_params=pltpu.CompilerParams(
        needs_layout_passes=False,
        skip_device_barrier=True,
        disable_bounds_checks=True,
        disable_semaphore_checks=True,
    ),
    out_type=jax.ShapeDtypeStruct((_B,), jnp.float32),
    scratch_types=[
        pltpu.VMEM((_BPW,), jnp.int32),
        pltpu.VMEM((_BPW, _G), jnp.float32),
        pltpu.VMEM((_BPW,), jnp.float32),
        pltpu.SemaphoreType.DMA,
    ],
)
def _depth_sc(xt_hbm, labels_hbm, out_hbm, lab_v, seg_v, out_v, sem):
    wid = lax.axis_index("s") * _NC + lax.axis_index("c")
    base = wid * _BPW

    pltpu.sync_copy(labels_hbm.at[pl.ds(base, _BPW)], lab_v)

    copies = [
        pltpu.make_async_copy(
            xt_hbm.at[
                plsc.Indices(lab_v.at[pl.ds(c * _G, _G)]),
                pl.ds(base + c * _G, _G),
            ],
            seg_v.at[pl.ds(c * _G, _G), :],
            sem,
        )
        for c in range(_NG)
    ]
    for cp in copies:
        cp.start()
    for cp in copies:
        cp.wait()

    lane = lax.iota(jnp.int32, _L)

    def extract(j, _):
        pos = j * _L + lane
        v = plsc.load_gather(seg_v, [pos, pos & (_G - 1)])
        out_v[pl.ds(j * _L, _L)] = jnp.exp(jnp.abs(v * jnp.float32(0.1))) - 1.0
        return _

    lax.fori_loop(0, _CHUNKS, extract, None)

    pltpu.sync_copy(out_v, out_hbm.at[pl.ds(base, _BPW)])


def kernel(x, labels):
    depth = _depth_sc(x.T, labels.astype(jnp.int32))
    return depth[:, None]
